# baseline (speedup 1.0000x reference)
import jax
import jax.numpy as jnp
from jax import lax
from jax.experimental import pallas as pl
from jax.experimental.pallas import tpu as pltpu
from jax.experimental.pallas import tpu_sc as plsc

HIDDEN_DIM = 128
MAX_DEPTH = 64
BATCH = 128
MAX_NODES = 256
VALUE_LEN = 32

NUM_CORES = 2
NUM_SUBCORES = 16
NUM_WORKERS = NUM_CORES * NUM_SUBCORES
LANES = 16

TOTAL_ROWS = BATCH * MAX_NODES
ROWS_PER_WORKER = TOTAL_ROWS // NUM_WORKERS
CHUNK = 8
NCH = ROWS_PER_WORKER // CHUNK
VCOLS = 128
VROWS = CHUNK * VALUE_LEN // VCOLS
SUBBLOCKS = ROWS_PER_WORKER * VALUE_LEN // VCOLS
IDXROWS = ROWS_PER_WORKER // LANES
NJ = HIDDEN_DIM // LANES


def _body(nt_hbm, dp_hbm, nv_hbm, node_tab, value_tab, depth_tab, out_hbm,
          nidx, didx, vidx, nb0, nb1, db0, db1, vb0, vb1, obuf, sm0, sm1):
    nbufs = (nb0, nb1)
    dbufs = (db0, db1)
    vbufs = (vb0, vb1)
    sems = (sm0, sm1)

    wid = lax.axis_index("s") * NUM_CORES + lax.axis_index("c")

    pltpu.sync_copy(nt_hbm.at[pl.ds(wid * IDXROWS, IDXROWS)], nidx)
    pltpu.sync_copy(dp_hbm.at[pl.ds(wid * IDXROWS, IDXROWS)], didx)
    pltpu.sync_copy(nv_hbm.at[pl.ds(wid * SUBBLOCKS, SUBBLOCKS)], vidx)

    def clamp_body(i, _):
        didx[i, :] = jnp.clip(didx[i, :], 0, MAX_DEPTH - 1)
        return 0
    lax.fori_loop(0, IDXROWS, clamp_body, 0)

    def fire(c, h):
        row = c // 2
        off = (c % 2) * CHUNK
        ds = [pltpu.async_copy(node_tab.at[nidx.at[row, pl.ds(off, CHUNK)]],
                               nbufs[h], sems[h]),
              pltpu.async_copy(depth_tab.at[didx.at[row, pl.ds(off, CHUNK)]],
                               dbufs[h], sems[h])]
        for k in range(VROWS):
            ds.append(pltpu.async_copy(
                value_tab.at[vidx.at[VROWS * c + k]],
                vbufs[h].at[pl.ds(k * VCOLS, VCOLS)], sems[h]))
        return ds

    scale = jnp.float32(1.0 / VALUE_LEN)

    def compute_store(c, h):
        vb = vbufs[h]
        for n in range(CHUNK):
            rowb = n * VALUE_LEN

            def l_body(l2, accs):
                l0 = 4 * l2
                for u in range(4):
                    accs = tuple(
                        accs[j] + vb[rowb + l0 + u, pl.ds(j * LANES, LANES)]
                        for j in range(NJ))
                return accs

            accs = tuple(jnp.zeros((LANES,), jnp.float32) for _ in range(NJ))
            accs = lax.fori_loop(0, VALUE_LEN // 4, l_body, accs)
            for j in range(NJ):
                obuf[n, pl.ds(j * LANES, LANES)] = (
                    accs[j] * scale
                    + nbufs[h][n, pl.ds(j * LANES, LANES)]
                    + dbufs[h][n, pl.ds(j * LANES, LANES)])
        pltpu.sync_copy(
            obuf, out_hbm.at[pl.ds(wid * ROWS_PER_WORKER + c * CHUNK, CHUNK)])

    for d in fire(0, 0):
        d.wait()

    def pair_body(i2, _):
        c0 = 2 * i2
        ds = fire(c0 + 1, 1)
        compute_store(c0, 0)
        for d in ds:
            d.wait()
        c1 = c0 + 1
        cn = jnp.minimum(c1 + 1, NCH - 1)
        ds2 = fire(cn, 0)
        compute_store(c1, 1)
        for d in ds2:
            d.wait()
        return 0

    lax.fori_loop(0, NCH // 2, pair_body, 0)


@jax.jit
def _tree_embedding(nt2, dp2, nv2, node_table, value_table, depth_table):
    mesh = plsc.VectorSubcoreMesh(core_axis_name="c", subcore_axis_name="s")
    return pl.kernel(
        _body,
        out_type=jax.ShapeDtypeStruct((TOTAL_ROWS, HIDDEN_DIM), jnp.float32),
        mesh=mesh,
        scratch_types=[
            pltpu.VMEM((IDXROWS, LANES), jnp.int32),
            pltpu.VMEM((IDXROWS, LANES), jnp.int32),
            pltpu.VMEM((SUBBLOCKS, VCOLS), jnp.int32),
            pltpu.VMEM((CHUNK, HIDDEN_DIM), jnp.float32),
            pltpu.VMEM((CHUNK, HIDDEN_DIM), jnp.float32),
            pltpu.VMEM((CHUNK, HIDDEN_DIM), jnp.float32),
            pltpu.VMEM((CHUNK, HIDDEN_DIM), jnp.float32),
            pltpu.VMEM((CHUNK * VALUE_LEN, HIDDEN_DIM), jnp.float32),
            pltpu.VMEM((CHUNK * VALUE_LEN, HIDDEN_DIM), jnp.float32),
            pltpu.VMEM((CHUNK, HIDDEN_DIM), jnp.float32),
            pltpu.SemaphoreType.DMA,
            pltpu.SemaphoreType.DMA,
        ],
    )(nt2, dp2, nv2, node_table, value_table, depth_table)


def kernel(node_types, node_values, depth, node_table, value_table, depth_table):
    nt2 = node_types.reshape(TOTAL_ROWS // LANES, LANES).astype(jnp.int32)
    dp2 = depth.reshape(TOTAL_ROWS // LANES, LANES).astype(jnp.int32)
    nv2 = node_values.reshape(TOTAL_ROWS * VALUE_LEN // VCOLS,
                              VCOLS).astype(jnp.int32)
    out = _tree_embedding(nt2, dp2, nv2, node_table, value_table, depth_table)
    return out.reshape(BATCH, MAX_NODES, HIDDEN_DIM)
